# Initial kernel scaffold; baseline (speedup 1.0000x reference)
import jax
import jax.numpy as jnp
from jax.experimental import pallas as pl


def kernel(user_id, age_bucket, gender, country, device, occupation, city, membership, table_user_id, table_age_bucket, table_gender, table_country, table_device, table_occupation, table_city, table_membership, W1, b1, W2, b2, W3, b3):
    raise NotImplementedError("write your pallas kernel here")


# speedup vs baseline: 1.5902x; 1.5902x over previous
import functools

import jax
import jax.numpy as jnp
from jax import lax
from jax.experimental import pallas as pl
from jax.experimental.pallas import tpu as pltpu
from jax.experimental.pallas import tpu_sc as plsc

_VOCABS = [100000, 100, 4, 256, 64, 128, 10000, 16]
_NF = 8
_B = 16384
_D = 128
_NC, _NS = 2, 16
_NW = _NC * _NS
_BPW = _B // _NW
_CHUNK = 128
_NCH = _BPW // _CHUNK


def _sc_gather(idx_all, tables):
    mesh = plsc.VectorSubcoreMesh(
        core_axis_name="c", subcore_axis_name="s",
        num_cores=_NC, num_subcores=_NS)

    @functools.partial(
        pl.kernel,
        out_type=jax.ShapeDtypeStruct((_NF, _B, _D), jnp.float32),
        mesh=mesh,
        scratch_types=[
            pltpu.VMEM((_NCH, _CHUNK), jnp.int32),
            pltpu.VMEM((_BPW, _D), jnp.float32),
            pltpu.SemaphoreType.DMA,
        ],
    )
    def k(idx_hbm, t0, t1, t2, t3, t4, t5, t6, t7, out_hbm,
          idx_v, rows_v, sem):
        wid = lax.axis_index("s") * _NC + lax.axis_index("c")
        tbls = [t0, t1, t2, t3, t4, t5, t6, t7]
        for f in range(_NF):
            pltpu.sync_copy(idx_hbm.at[f, pl.ds(wid * _NCH, _NCH)], idx_v)
            copies = [
                pltpu.async_copy(
                    tbls[f].at[idx_v.at[c]],
                    rows_v.at[pl.ds(c * _CHUNK, _CHUNK)],
                    sem)
                for c in range(_NCH)
            ]
            for cp in copies:
                cp.wait()
            pltpu.sync_copy(rows_v, out_hbm.at[f, pl.ds(wid * _BPW, _BPW)])

    return k(idx_all, *tables)


def _mlp(xg, W1, b1, W2, b2, W3, b3, block_b=2048):
    def body(xg_ref, w1_ref, b1_ref, w2_ref, b2_ref, w3_ref, b3_ref,
             out_ref):
        acc = None
        for f in range(_NF):
            p = lax.dot_general(
                xg_ref[f], w1_ref[:, f * _D:(f + 1) * _D],
                (((1,), (1,)), ((), ())),
                preferred_element_type=jnp.float32)
            acc = p if acc is None else acc + p
        h1 = jnp.maximum(acc + b1_ref[...], 0.0)
        h2 = jnp.maximum(
            lax.dot_general(h1, w2_ref[...], (((1,), (1,)), ((), ())),
                            preferred_element_type=jnp.float32)
            + b2_ref[...], 0.0)
        o = lax.dot_general(h2, w3_ref[...], (((1,), (1,)), ((), ())),
                            preferred_element_type=jnp.float32) + b3_ref[...]
        n2 = jnp.sum(o * o, axis=1, keepdims=True)
        out_ref[...] = o * lax.rsqrt(jnp.maximum(n2, 1e-24))

    din = _NF * _D
    h1d, h2d = W1.shape[0], W2.shape[0]
    return pl.pallas_call(
        body,
        grid=(_B // block_b,),
        in_specs=[
            pl.BlockSpec((_NF, block_b, _D), lambda i: (0, i, 0)),
            pl.BlockSpec((h1d, din), lambda i: (0, 0)),
            pl.BlockSpec((1, h1d), lambda i: (0, 0)),
            pl.BlockSpec((h2d, h1d), lambda i: (0, 0)),
            pl.BlockSpec((1, h2d), lambda i: (0, 0)),
            pl.BlockSpec((_D, h2d), lambda i: (0, 0)),
            pl.BlockSpec((1, _D), lambda i: (0, 0)),
        ],
        out_specs=pl.BlockSpec((block_b, _D), lambda i: (i, 0)),
        out_shape=jax.ShapeDtypeStruct((_B, _D), jnp.float32),
    )(xg, W1, b1.reshape(1, -1), W2, b2.reshape(1, -1), W3,
      b3.reshape(1, -1))


def kernel(user_id, age_bucket, gender, country, device, occupation, city,
           membership, table_user_id, table_age_bucket, table_gender,
           table_country, table_device, table_occupation, table_city,
           table_membership, W1, b1, W2, b2, W3, b3):
    idxs = [user_id, age_bucket, gender, country, device, occupation, city,
            membership]
    tables = [table_user_id, table_age_bucket, table_gender, table_country,
              table_device, table_occupation, table_city, table_membership]
    clipped = [jnp.clip(i, 0, v - 1) for i, v in zip(idxs, _VOCABS)]
    idx_all = jnp.stack(clipped).reshape(_NF, _B // _CHUNK, _CHUNK)
    xg = _sc_gather(idx_all, tables)
    return _mlp(xg, W1, b1, W2, b2, W3, b3)


# speedup vs baseline: 1.5979x; 1.0049x over previous
import functools

import jax
import jax.numpy as jnp
from jax import lax
from jax.experimental import pallas as pl
from jax.experimental.pallas import tpu as pltpu
from jax.experimental.pallas import tpu_sc as plsc

_VOCABS = [100000, 100, 4, 256, 64, 128, 10000, 16]
_NF = 8
_B = 16384
_D = 128
_NC, _NS = 2, 16
_NW = _NC * _NS
_BPW = _B // _NW
_CHUNK = 128
_NCH = _BPW // _CHUNK


_HALF = _BPW // 2
_NU = _NF * 2


def _sc_gather(idx_all, tables):
    mesh = plsc.VectorSubcoreMesh(
        core_axis_name="c", subcore_axis_name="s",
        num_cores=_NC, num_subcores=_NS)

    @functools.partial(
        pl.kernel,
        out_type=jax.ShapeDtypeStruct((_NF, _B, _D), jnp.float32),
        mesh=mesh,
        scratch_types=[
            pltpu.VMEM((_NF, _NCH, _CHUNK), jnp.int32),
            pltpu.VMEM((2, _HALF, _D), jnp.float32),
            pltpu.SemaphoreType.DMA,
            pltpu.SemaphoreType.DMA,
            pltpu.SemaphoreType.DMA,
        ],
    )
    def k(idx_hbm, t0, t1, t2, t3, t4, t5, t6, t7, out_hbm,
          idx_v, rows_v, gsem0, gsem1, wsem):
        wid = lax.axis_index("s") * _NC + lax.axis_index("c")
        tbls = [t0, t1, t2, t3, t4, t5, t6, t7]
        gsems = [gsem0, gsem1]
        pltpu.sync_copy(idx_hbm.at[wid], idx_v)

        gathers = [None] * _NU
        wbs = [None] * _NU

        def fire_gather(u):
            f, half = u // 2, u % 2
            buf = u % 2
            gathers[u] = [
                pltpu.async_copy(
                    tbls[f].at[idx_v.at[f, 2 * half + c]],
                    rows_v.at[buf, pl.ds(c * _CHUNK, _CHUNK)],
                    gsems[buf])
                for c in range(2)
            ]

        def fire_wb(u):
            f, half = u // 2, u % 2
            buf = u % 2
            wbs[u] = pltpu.async_copy(
                rows_v.at[buf],
                out_hbm.at[f, pl.ds(wid * _BPW + half * _HALF, _HALF)],
                wsem)

        fire_gather(0)
        for u in range(_NU):
            if u + 1 < _NU:
                if u >= 1:
                    wbs[u - 1].wait()
                fire_gather(u + 1)
            for cp in gathers[u]:
                cp.wait()
            fire_wb(u)
        wbs[_NU - 2].wait()
        wbs[_NU - 1].wait()

    return k(idx_all, *tables)


def _mlp(xg, W1, b1, W2, b2, W3, b3, block_b=2048):
    def body(xg_ref, w1_ref, b1_ref, w2_ref, b2_ref, w3_ref, b3_ref,
             out_ref):
        acc = None
        for f in range(_NF):
            p = lax.dot_general(
                xg_ref[f], w1_ref[:, f * _D:(f + 1) * _D],
                (((1,), (1,)), ((), ())),
                preferred_element_type=jnp.float32)
            acc = p if acc is None else acc + p
        h1 = jnp.maximum(acc + b1_ref[...], 0.0)
        h2 = jnp.maximum(
            lax.dot_general(h1, w2_ref[...], (((1,), (1,)), ((), ())),
                            preferred_element_type=jnp.float32)
            + b2_ref[...], 0.0)
        o = lax.dot_general(h2, w3_ref[...], (((1,), (1,)), ((), ())),
                            preferred_element_type=jnp.float32) + b3_ref[...]
        n2 = jnp.sum(o * o, axis=1, keepdims=True)
        out_ref[...] = o * lax.rsqrt(jnp.maximum(n2, 1e-24))

    din = _NF * _D
    h1d, h2d = W1.shape[0], W2.shape[0]
    return pl.pallas_call(
        body,
        grid=(_B // block_b,),
        in_specs=[
            pl.BlockSpec((_NF, block_b, _D), lambda i: (0, i, 0)),
            pl.BlockSpec((h1d, din), lambda i: (0, 0)),
            pl.BlockSpec((1, h1d), lambda i: (0, 0)),
            pl.BlockSpec((h2d, h1d), lambda i: (0, 0)),
            pl.BlockSpec((1, h2d), lambda i: (0, 0)),
            pl.BlockSpec((_D, h2d), lambda i: (0, 0)),
            pl.BlockSpec((1, _D), lambda i: (0, 0)),
        ],
        out_specs=pl.BlockSpec((block_b, _D), lambda i: (i, 0)),
        out_shape=jax.ShapeDtypeStruct((_B, _D), jnp.float32),
    )(xg, W1, b1.reshape(1, -1), W2, b2.reshape(1, -1), W3,
      b3.reshape(1, -1))


def kernel(user_id, age_bucket, gender, country, device, occupation, city,
           membership, table_user_id, table_age_bucket, table_gender,
           table_country, table_device, table_occupation, table_city,
           table_membership, W1, b1, W2, b2, W3, b3):
    idxs = [user_id, age_bucket, gender, country, device, occupation, city,
            membership]
    tables = [table_user_id, table_age_bucket, table_gender, table_country,
              table_device, table_occupation, table_city, table_membership]
    clipped = [jnp.clip(i, 0, v - 1) for i, v in zip(idxs, _VOCABS)]
    idx_all = jnp.stack(clipped).reshape(
        _NF, _NW, _NCH, _CHUNK).transpose(1, 0, 2, 3)
    xg = _sc_gather(idx_all, tables)
    return _mlp(xg, W1, b1, W2, b2, W3, b3)


# speedup vs baseline: 7.2878x; 4.5608x over previous
import functools

import jax
import jax.numpy as jnp
from jax import lax
from jax.experimental import pallas as pl
from jax.experimental.pallas import tpu as pltpu
from jax.experimental.pallas import tpu_sc as plsc

_NF = 8
_B = 16384
_D = 128
_NC, _NS = 2, 16
_NW = _NC * _NS
_BPW = _B // _NW
_CHUNK = 128
_NCH = _BPW // _CHUNK
_HALF = _BPW // 2

_VOCABS = [100000, 100, 4, 256, 64, 128, 10000, 16]
_BIG = [0, 6]
_SMALL = [1, 2, 3, 4, 5, 7]
_SPAD = [(v + 7) // 8 * 8 for v in (_VOCABS[f] for f in _SMALL)]
_SOFF = [sum(_SPAD[:i]) for i in range(len(_SPAD))]
_KS = sum(_SPAD)


def _sc_gather(idx_all, t_user, t_city):
    mesh = plsc.VectorSubcoreMesh(
        core_axis_name="c", subcore_axis_name="s",
        num_cores=_NC, num_subcores=_NS)

    nu = 4

    @functools.partial(
        pl.kernel,
        out_type=jax.ShapeDtypeStruct((2, _B, _D), jnp.float32),
        mesh=mesh,
        scratch_types=[
            pltpu.VMEM((2, _NCH, _CHUNK), jnp.int32),
            pltpu.VMEM((2, _HALF, _D), jnp.float32),
            pltpu.SemaphoreType.DMA,
            pltpu.SemaphoreType.DMA,
            pltpu.SemaphoreType.DMA,
        ],
    )
    def k(idx_hbm, t0, t1, out_hbm, idx_v, rows_v, gsem0, gsem1, wsem):
        wid = lax.axis_index("s") * _NC + lax.axis_index("c")
        tbls = [t0, t1]
        gsems = [gsem0, gsem1]
        pltpu.sync_copy(idx_hbm.at[wid], idx_v)

        gathers = [None] * nu
        wbs = [None] * nu

        def fire_gather(u):
            f, half = u // 2, u % 2
            buf = u % 2
            gathers[u] = [
                pltpu.async_copy(
                    tbls[f].at[idx_v.at[f, 2 * half + c]],
                    rows_v.at[buf, pl.ds(c * _CHUNK, _CHUNK)],
                    gsems[buf])
                for c in range(2)
            ]

        def fire_wb(u):
            f, half = u // 2, u % 2
            buf = u % 2
            wbs[u] = pltpu.async_copy(
                rows_v.at[buf],
                out_hbm.at[f, pl.ds(wid * _BPW + half * _HALF, _HALF)],
                wsem)

        fire_gather(0)
        for u in range(nu):
            if u + 1 < nu:
                if u >= 1:
                    wbs[u - 1].wait()
                fire_gather(u + 1)
            for cp in gathers[u]:
                cp.wait()
            fire_wb(u)
        wbs[nu - 2].wait()
        wbs[nu - 1].wait()

    return k(idx_all, t_user, t_city)


def _mlp(xg2, tgt8, ts, W1, b1, W2, b2, W3, b3, block_b=2048):
    h1d, h2d = W1.shape[0], W2.shape[0]
    din = _NF * _D

    def body(xg_ref, tgt_ref, ts_ref, w1_ref, b1_ref, w2_ref, b2_ref,
             w3_ref, b3_ref, out_ref, m_ref):
        @pl.when(pl.program_id(0) == 0)
        def _():
            for (f, off, pv) in zip(_SMALL, _SOFF, _SPAD):
                m_ref[pl.ds(off, pv), :] = lax.dot_general(
                    ts_ref[pl.ds(off, pv), :],
                    w1_ref[:, f * _D:(f + 1) * _D],
                    (((1,), (1,)), ((), ())),
                    preferred_element_type=jnp.float32)

        cols = lax.broadcasted_iota(jnp.int32, (block_b, _KS), 1)
        oh = None
        for i in range(len(_SMALL)):
            m = (cols == tgt_ref[i][:, None]).astype(jnp.float32)
            oh = m if oh is None else oh + m
        acc = lax.dot_general(oh, m_ref[...], (((1,), (0,)), ((), ())),
                              preferred_element_type=jnp.float32)
        for g, f in enumerate(_BIG):
            acc = acc + lax.dot_general(
                xg_ref[g], w1_ref[:, f * _D:(f + 1) * _D],
                (((1,), (1,)), ((), ())),
                preferred_element_type=jnp.float32)
        h1 = jnp.maximum(acc + b1_ref[...], 0.0)
        h2 = jnp.maximum(
            lax.dot_general(h1, w2_ref[...], (((1,), (1,)), ((), ())),
                            preferred_element_type=jnp.float32)
            + b2_ref[...], 0.0)
        o = lax.dot_general(h2, w3_ref[...], (((1,), (1,)), ((), ())),
                            preferred_element_type=jnp.float32) + b3_ref[...]
        n2 = jnp.sum(o * o, axis=1, keepdims=True)
        out_ref[...] = o * lax.rsqrt(jnp.maximum(n2, 1e-24))

    return pl.pallas_call(
        body,
        grid=(_B // block_b,),
        in_specs=[
            pl.BlockSpec((2, block_b, _D), lambda i: (0, i, 0)),
            pl.BlockSpec((8, block_b), lambda i: (0, i)),
            pl.BlockSpec((_KS, _D), lambda i: (0, 0)),
            pl.BlockSpec((h1d, din), lambda i: (0, 0)),
            pl.BlockSpec((1, h1d), lambda i: (0, 0)),
            pl.BlockSpec((h2d, h1d), lambda i: (0, 0)),
            pl.BlockSpec((1, h2d), lambda i: (0, 0)),
            pl.BlockSpec((_D, h2d), lambda i: (0, 0)),
            pl.BlockSpec((1, _D), lambda i: (0, 0)),
        ],
        out_specs=pl.BlockSpec((block_b, _D), lambda i: (i, 0)),
        out_shape=jax.ShapeDtypeStruct((_B, _D), jnp.float32),
        scratch_shapes=[pltpu.VMEM((_KS, h1d), jnp.float32)],
    )(xg2, tgt8, ts, W1, b1.reshape(1, -1), W2, b2.reshape(1, -1), W3,
      b3.reshape(1, -1))


def kernel(user_id, age_bucket, gender, country, device, occupation, city,
           membership, table_user_id, table_age_bucket, table_gender,
           table_country, table_device, table_occupation, table_city,
           table_membership, W1, b1, W2, b2, W3, b3):
    idxs = [user_id, age_bucket, gender, country, device, occupation, city,
            membership]
    tables = [table_user_id, table_age_bucket, table_gender, table_country,
              table_device, table_occupation, table_city, table_membership]
    clipped = [jnp.clip(i, 0, v - 1) for i, v in zip(idxs, _VOCABS)]

    idx_big = jnp.stack([clipped[f] for f in _BIG]).reshape(
        2, _NW, _NCH, _CHUNK).transpose(1, 0, 2, 3)
    xg2 = _sc_gather(idx_big, tables[_BIG[0]], tables[_BIG[1]])

    tgt = [clipped[f] + off for f, off in zip(_SMALL, _SOFF)]
    tgt8 = jnp.stack(tgt + [jnp.full((_B,), -1, jnp.int32)] * 2)

    ts = jnp.concatenate([
        jnp.pad(tables[f], ((0, pv - tables[f].shape[0]), (0, 0)))
        for f, pv in zip(_SMALL, _SPAD)
    ], axis=0)

    return _mlp(xg2, tgt8, ts, W1, b1, W2, b2, W3, b3)
